# baseline (speedup 1.0000x reference)
import functools

import jax
import jax.numpy as jnp
import numpy as np
from jax.experimental import pallas as pl
from jax.experimental.pallas import tpu as pltpu

_MARGINS = np.array(
    [0.45357266, 1.0, 0.49222963, 0.76696184, 1.0, 0.43823621, 0.60325897,
     0.57481898],
    dtype=np.float32,
)
_M = (2.4 * _MARGINS).astype(np.float32)
_C = 8
_G = 16


def _body(x_ref, tgt_ref, msk_ref, sum_ref, cnt_ref):
    i = pl.program_id(0)

    @pl.when(i == 0)
    def _init():
        sum_ref[0, 0] = jnp.float32(0.0)
        cnt_ref[0, 0] = jnp.float32(0.0)

    x = x_ref[...]
    tgt = tgt_ref[...]
    mskf = msk_ref[...]
    B = x.shape[0]

    s_of_lane = jax.lax.broadcasted_iota(jnp.int32, (_G, 128), 1) // _C
    s_idx = jax.lax.broadcasted_iota(jnp.int32, (_G, 128), 0)
    R = (s_of_lane == s_idx).astype(jnp.float32)
    T = jax.lax.dot(tgt.astype(jnp.float32), R,
                    preferred_element_type=jnp.float32)

    cls_i = jax.lax.broadcasted_iota(jnp.int32, (B, 128), 1) % _C
    onehot = (T == cls_i.astype(jnp.float32))
    m_lane = jnp.zeros((B, 128), jnp.float32)
    for c in range(_C):
        m_lane = jnp.where(cls_i == c, jnp.float32(_M[c]), m_lane)
    out = jnp.where(onehot, x - m_lane, x)

    l_idx = jax.lax.broadcasted_iota(jnp.int32, (128, _G), 0)
    s_col = jax.lax.broadcasted_iota(jnp.int32, (128, _G), 1)
    A = (l_idx // _C == s_col).astype(jnp.float32)

    e = jnp.exp(out)
    S = jax.lax.dot(e, A, preferred_element_type=jnp.float32)
    gold = jax.lax.dot(jnp.where(onehot, out, 0.0), A,
                       preferred_element_type=jnp.float32)
    per = jnp.log(S) - gold

    sum_ref[0, 0] += jnp.sum(per * mskf)
    cnt_ref[0, 0] += jnp.sum(mskf)


@jax.jit
def kernel(x, target, mask):
    N, C = x.shape
    assert C == _C
    rows = N // _G
    xr = x.reshape(rows, _G * _C)
    tgt = target.reshape(rows, _G)
    mskf = mask.astype(jnp.float32).reshape(rows, _G)

    B = 1024
    grid = (rows // B,)
    out_shape = [
        jax.ShapeDtypeStruct((1, 1), jnp.float32),
        jax.ShapeDtypeStruct((1, 1), jnp.float32),
    ]
    s, c = pl.pallas_call(
        _body,
        grid=grid,
        in_specs=[
            pl.BlockSpec((B, _G * _C), lambda i: (i, 0)),
            pl.BlockSpec((B, _G), lambda i: (i, 0)),
            pl.BlockSpec((B, _G), lambda i: (i, 0)),
        ],
        out_specs=[
            pl.BlockSpec(memory_space=pltpu.SMEM),
            pl.BlockSpec(memory_space=pltpu.SMEM),
        ],
        out_shape=out_shape,
        compiler_params=pltpu.CompilerParams(
            dimension_semantics=("arbitrary",),
        ),
    )(xr, tgt, mskf)
    return (s[0, 0] / c[0, 0]).astype(jnp.float32)


# speedup vs baseline: 8.3201x; 8.3201x over previous
import jax
import jax.numpy as jnp
import numpy as np
from jax.experimental import pallas as pl
from jax.experimental.pallas import tpu as pltpu

_MARGINS = np.array(
    [0.45357266, 1.0, 0.49222963, 0.76696184, 1.0, 0.43823621, 0.60325897,
     0.57481898],
    dtype=np.float32,
)
_M = (2.4 * _MARGINS).astype(np.float32)
_C = 8


def _body(x_ref, tgt_ref, msk_ref, sum_ref, cnt_ref):
    i = pl.program_id(0)

    @pl.when(i == 0)
    def _init():
        sum_ref[0, 0] = jnp.float32(0.0)
        cnt_ref[0, 0] = jnp.float32(0.0)

    x = x_ref[...]
    tgt = tgt_ref[...]
    mskf = msk_ref[...].astype(jnp.float32)
    Bb = x.shape[0]

    m_t = jnp.zeros(tgt.shape, jnp.float32)
    for c in range(_C):
        m_t = jnp.where(tgt == c, jnp.float32(_M[c]), m_t)

    cls = jax.lax.broadcasted_iota(jnp.int32, (Bb, _C, 128), 1)
    onehot = cls == tgt[:, None, :]
    out = jnp.where(onehot, x - m_t[:, None, :], x)

    e = jnp.exp(out)
    S = jnp.sum(e, axis=1)
    gold = jnp.sum(jnp.where(onehot, out, 0.0), axis=1)
    per = jnp.log(S) - gold

    sum_ref[0, 0] += jnp.sum(per * mskf)
    cnt_ref[0, 0] += jnp.sum(mskf)


@jax.jit
def kernel(x, target, mask):
    N, C = x.shape
    assert C == _C
    rows = N // 128
    xv = x.reshape(rows, 128, C).transpose(0, 2, 1)
    tgt = target.reshape(rows, 128)
    msk = mask.reshape(rows, 128)

    Bb = 256
    grid = (rows // Bb,)
    out_shape = [
        jax.ShapeDtypeStruct((1, 1), jnp.float32),
        jax.ShapeDtypeStruct((1, 1), jnp.float32),
    ]
    s, c = pl.pallas_call(
        _body,
        grid=grid,
        in_specs=[
            pl.BlockSpec((Bb, C, 128), lambda i: (i, 0, 0)),
            pl.BlockSpec((Bb, 128), lambda i: (i, 0)),
            pl.BlockSpec((Bb, 128), lambda i: (i, 0)),
        ],
        out_specs=[
            pl.BlockSpec(memory_space=pltpu.SMEM),
            pl.BlockSpec(memory_space=pltpu.SMEM),
        ],
        out_shape=out_shape,
        compiler_params=pltpu.CompilerParams(
            dimension_semantics=("arbitrary",),
        ),
    )(xv, tgt, msk)
    return (s[0, 0] / c[0, 0]).astype(jnp.float32)


# speedup vs baseline: 12.5216x; 1.5050x over previous
import jax
import jax.numpy as jnp
import numpy as np
from jax.experimental import pallas as pl
from jax.experimental.pallas import tpu as pltpu

_MARGINS = np.array(
    [0.45357266, 1.0, 0.49222963, 0.76696184, 1.0, 0.43823621, 0.60325897,
     0.57481898],
    dtype=np.float32,
)
_M = (2.4 * _MARGINS).astype(np.float32)
_C = 8


def _body(x_ref, tgt_ref, msk_ref, sum_ref, cnt_ref):
    i = pl.program_id(0)

    @pl.when(i == 0)
    def _init():
        sum_ref[0, 0] = jnp.float32(0.0)
        cnt_ref[0, 0] = jnp.float32(0.0)

    x = x_ref[...]
    tgt = tgt_ref[...]
    mskf = msk_ref[...].astype(jnp.float32)
    Bb = x.shape[0]

    xt = jnp.transpose(x, (1, 0, 2))
    S = jnp.zeros((Bb, 128), jnp.float32)
    gacc = jnp.zeros((Bb, 128), jnp.float32)
    for c in range(_C):
        xc = xt[c]
        sel = tgt == c
        xm = xc - jnp.float32(_M[c])
        S = S + jnp.exp(jnp.where(sel, xm, xc))
        gacc = gacc + jnp.where(sel, xm, 0.0)

    sum_ref[0, 0] += jnp.sum(mskf * (jnp.log(S) - gacc))
    cnt_ref[0, 0] += jnp.sum(mskf)


@jax.jit
def kernel(x, target, mask):
    N, C = x.shape
    assert C == _C
    rows = N // 128
    xv = x.reshape(rows, 128, C).transpose(0, 2, 1)
    tgt = target.reshape(rows, 128)
    msk = mask.reshape(rows, 128)

    Bb = 256
    grid = (rows // Bb,)
    out_shape = [
        jax.ShapeDtypeStruct((1, 1), jnp.float32),
        jax.ShapeDtypeStruct((1, 1), jnp.float32),
    ]
    s, c = pl.pallas_call(
        _body,
        grid=grid,
        in_specs=[
            pl.BlockSpec((Bb, C, 128), lambda i: (i, 0, 0)),
            pl.BlockSpec((Bb, 128), lambda i: (i, 0)),
            pl.BlockSpec((Bb, 128), lambda i: (i, 0)),
        ],
        out_specs=[
            pl.BlockSpec(memory_space=pltpu.SMEM),
            pl.BlockSpec(memory_space=pltpu.SMEM),
        ],
        out_shape=out_shape,
        compiler_params=pltpu.CompilerParams(
            dimension_semantics=("arbitrary",),
        ),
    )(xv, tgt, msk)
    return (s[0, 0] / c[0, 0]).astype(jnp.float32)
